# baseline (speedup 1.0000x reference)
import functools

import jax
import jax.numpy as jnp
from jax import lax
from jax.experimental import pallas as pl
from jax.experimental.pallas import tpu as pltpu
from jax.experimental.pallas import tpu_sc as plsc

B = 1024
N_SPECIES = 1000
N_REAC = 20000
INV_DTG = 100.0

NC = 2
NSUB = 16
NW = NC * NSUB
ROWS = B // NW
LANES = 16
CHUNK = 512
LAST = N_REAC % CHUNK
CHUNK_STARTS = tuple(range(0, N_REAC - LAST, CHUNK))


def _sc_gather_body(rh_hbm, i0_hbm, i1_hbm, out_hbm, rh_flat, i0_v, i1_v,
                    ob_a, ob_b, ob_last_v, sem_a, sem_b):
    wid = lax.axis_index("s") * NC + lax.axis_index("c")
    base = wid * ROWS
    pltpu.sync_copy(rh_hbm.at[pl.ds(base * N_SPECIES, ROWS * N_SPECIES)], rh_flat)
    pltpu.sync_copy(i0_hbm, i0_v)
    pltpu.sync_copy(i1_hbm, i1_v)

    GRP = 32

    def make_body(c0, buf):
        def j_body(j, carry):
            r0 = c0 + j * LANES
            idx0 = i0_v[pl.ds(r0, LANES)]
            idx1 = i1_v[pl.ds(r0, LANES)]
            for g in range(0, ROWS, GRP):
                pairs = []
                for bl in range(g, g + GRP):
                    v0 = plsc.load_gather(rh_flat, [idx0 + bl * N_SPECIES])
                    v1 = plsc.load_gather(rh_flat, [idx1 + bl * N_SPECIES])
                    pairs.append(v0 + v1)
                for bl, v in zip(range(g, g + GRP), pairs):
                    buf[bl, pl.ds(j * LANES, LANES)] = v
            return carry
        return j_body

    bufs = (ob_a, ob_b)
    sems = (sem_a, sem_b)
    descs = [None, None]
    for ci, c0 in enumerate(CHUNK_STARTS):
        p = ci % 2
        if descs[p] is not None:
            descs[p].wait()
        lax.fori_loop(0, CHUNK // LANES, make_body(c0, bufs[p]), 0)
        descs[p] = pltpu.async_copy(
            bufs[p], out_hbm.at[pl.ds(base, ROWS), pl.ds(c0, CHUNK)], sems[p]
        )

    c0 = N_REAC - LAST
    lax.fori_loop(0, LAST // LANES, make_body(c0, ob_last_v), 0)
    pltpu.sync_copy(ob_last_v, out_hbm.at[pl.ds(base, ROWS), pl.ds(c0, LAST)])
    for d in descs:
        d.wait()


@functools.cache
def _sc_gather_kernel():
    return pl.kernel(
        _sc_gather_body,
        out_type=jax.ShapeDtypeStruct((B, N_REAC), jnp.float32),
        mesh=plsc.VectorSubcoreMesh(
            core_axis_name="c", subcore_axis_name="s",
            num_cores=NC, num_subcores=NSUB,
        ),
        scratch_types=[
            pltpu.VMEM((ROWS * N_SPECIES,), jnp.float32),
            pltpu.VMEM((N_REAC,), jnp.int32),
            pltpu.VMEM((N_REAC,), jnp.int32),
            pltpu.VMEM((ROWS, CHUNK), jnp.float32),
            pltpu.VMEM((ROWS, CHUNK), jnp.float32),
            pltpu.VMEM((ROWS, LAST), jnp.float32),
            pltpu.SemaphoreType.DMA,
            pltpu.SemaphoreType.DMA,
        ],
        compiler_params=pltpu.CompilerParams(needs_layout_passes=False),
        name="sc_gather_sum",
    )


BBLK = 128


def _tc_elemwise_body(rh_ref, t_ref, g_ref, ea_ref, lt_ref, al_ref, br_ref, o_ref):
    inv_t = 1.0 / t_ref[...]
    scale = INV_DTG / g_ref[...]
    lp = jnp.maximum(-ea_ref[...] * inv_t, lt_ref[...])
    coef = al_ref[...] * br_ref[...]
    o_ref[...] = (coef * scale) * rh_ref[...] * jnp.exp(lp)


def _tc_elemwise(rate_hop, t_dust, den_gas, ea, lt, al, br):
    grid = (B // BBLK,)
    pspec = pl.BlockSpec((1, N_REAC), lambda i: (0, 0))
    return pl.pallas_call(
        _tc_elemwise_body,
        grid=grid,
        in_specs=[
            pl.BlockSpec((BBLK, N_REAC), lambda i: (i, 0)),
            pl.BlockSpec((BBLK, 1), lambda i: (i, 0)),
            pl.BlockSpec((BBLK, 1), lambda i: (i, 0)),
            pspec, pspec, pspec, pspec,
        ],
        out_specs=pl.BlockSpec((BBLK, N_REAC), lambda i: (i, 0)),
        out_shape=jax.ShapeDtypeStruct((B, N_REAC), jnp.float32),
        compiler_params=pltpu.CompilerParams(vmem_limit_bytes=100 * 1024 * 1024),
    )(rate_hop, t_dust, den_gas, ea, lt, al, br)


@jax.jit
def kernel(rate_hopping, T_dust, den_gas, E_act, log_prob_surf_tunl, alpha,
           branching_ratio, inds_r):
    i0 = inds_r[:, 0]
    i1 = inds_r[:, 1]
    rate_hop = _sc_gather_kernel()(rate_hopping.reshape(B * N_SPECIES), i0, i1)
    return _tc_elemwise(
        rate_hop, T_dust, den_gas,
        E_act.reshape(1, N_REAC),
        log_prob_surf_tunl.reshape(1, N_REAC),
        alpha.reshape(1, N_REAC),
        branching_ratio.reshape(1, N_REAC),
    )


# speedup vs baseline: 1.0040x; 1.0040x over previous
import functools

import jax
import jax.numpy as jnp
from jax import lax
from jax.experimental import pallas as pl
from jax.experimental.pallas import tpu as pltpu
from jax.experimental.pallas import tpu_sc as plsc

B = 1024
N_SPECIES = 1000
N_REAC = 20000
INV_DTG = 100.0

NC = 2
NSUB = 16
NW = NC * NSUB
ROWS = B // NW
LANES = 16
CHUNK = 512
LAST = N_REAC % CHUNK
CHUNK_STARTS = tuple(range(0, N_REAC - LAST, CHUNK))


def _sc_gather_body(rh_hbm, i0_hbm, i1_hbm, out_hbm, rh_flat, i0_v, i1_v,
                    ob_a, ob_b, ob_last_v, sem_a, sem_b):
    wid = lax.axis_index("s") * NC + lax.axis_index("c")
    base = wid * ROWS
    pltpu.sync_copy(rh_hbm.at[pl.ds(base * N_SPECIES, ROWS * N_SPECIES)], rh_flat)
    pltpu.sync_copy(i0_hbm, i0_v)
    pltpu.sync_copy(i1_hbm, i1_v)

    GRP = 16

    def make_body(c0, buf):
        def j_body(j, carry):
            r0 = c0 + j * LANES
            idx0 = i0_v[pl.ds(r0, LANES)]
            idx1 = i1_v[pl.ds(r0, LANES)]
            for g in range(0, ROWS, GRP):
                pairs = []
                for bl in range(g, g + GRP):
                    v0 = plsc.load_gather(rh_flat, [idx0 + bl * N_SPECIES])
                    v1 = plsc.load_gather(rh_flat, [idx1 + bl * N_SPECIES])
                    pairs.append(v0 + v1)
                for bl, v in zip(range(g, g + GRP), pairs):
                    buf[bl, pl.ds(j * LANES, LANES)] = v
            return carry
        return j_body

    bufs = (ob_a, ob_b)
    sems = (sem_a, sem_b)
    descs = [None, None]
    for ci, c0 in enumerate(CHUNK_STARTS):
        p = ci % 2
        if descs[p] is not None:
            descs[p].wait()
        lax.fori_loop(0, CHUNK // LANES, make_body(c0, bufs[p]), 0)
        descs[p] = pltpu.async_copy(
            bufs[p], out_hbm.at[pl.ds(base, ROWS), pl.ds(c0, CHUNK)], sems[p]
        )

    c0 = N_REAC - LAST
    lax.fori_loop(0, LAST // LANES, make_body(c0, ob_last_v), 0)
    pltpu.sync_copy(ob_last_v, out_hbm.at[pl.ds(base, ROWS), pl.ds(c0, LAST)])
    for d in descs:
        d.wait()


@functools.cache
def _sc_gather_kernel():
    return pl.kernel(
        _sc_gather_body,
        out_type=jax.ShapeDtypeStruct((B, N_REAC), jnp.float32),
        mesh=plsc.VectorSubcoreMesh(
            core_axis_name="c", subcore_axis_name="s",
            num_cores=NC, num_subcores=NSUB,
        ),
        scratch_types=[
            pltpu.VMEM((ROWS * N_SPECIES,), jnp.float32),
            pltpu.VMEM((N_REAC,), jnp.int32),
            pltpu.VMEM((N_REAC,), jnp.int32),
            pltpu.VMEM((ROWS, CHUNK), jnp.float32),
            pltpu.VMEM((ROWS, CHUNK), jnp.float32),
            pltpu.VMEM((ROWS, LAST), jnp.float32),
            pltpu.SemaphoreType.DMA,
            pltpu.SemaphoreType.DMA,
        ],
        compiler_params=pltpu.CompilerParams(needs_layout_passes=False),
        name="sc_gather_sum",
    )


BBLK = 128


def _tc_elemwise_body(rh_ref, t_ref, g_ref, ea_ref, lt_ref, al_ref, br_ref, o_ref):
    inv_t = 1.0 / t_ref[...]
    scale = INV_DTG / g_ref[...]
    lp = jnp.maximum(-ea_ref[...] * inv_t, lt_ref[...])
    coef = al_ref[...] * br_ref[...]
    o_ref[...] = (coef * scale) * rh_ref[...] * jnp.exp(lp)


def _tc_elemwise(rate_hop, t_dust, den_gas, ea, lt, al, br):
    grid = (B // BBLK,)
    pspec = pl.BlockSpec((1, N_REAC), lambda i: (0, 0))
    return pl.pallas_call(
        _tc_elemwise_body,
        grid=grid,
        in_specs=[
            pl.BlockSpec((BBLK, N_REAC), lambda i: (i, 0)),
            pl.BlockSpec((BBLK, 1), lambda i: (i, 0)),
            pl.BlockSpec((BBLK, 1), lambda i: (i, 0)),
            pspec, pspec, pspec, pspec,
        ],
        out_specs=pl.BlockSpec((BBLK, N_REAC), lambda i: (i, 0)),
        out_shape=jax.ShapeDtypeStruct((B, N_REAC), jnp.float32),
        compiler_params=pltpu.CompilerParams(vmem_limit_bytes=100 * 1024 * 1024),
    )(rate_hop, t_dust, den_gas, ea, lt, al, br)


@jax.jit
def kernel(rate_hopping, T_dust, den_gas, E_act, log_prob_surf_tunl, alpha,
           branching_ratio, inds_r):
    i0 = inds_r[:, 0]
    i1 = inds_r[:, 1]
    rate_hop = _sc_gather_kernel()(rate_hopping.reshape(B * N_SPECIES), i0, i1)
    return _tc_elemwise(
        rate_hop, T_dust, den_gas,
        E_act.reshape(1, N_REAC),
        log_prob_surf_tunl.reshape(1, N_REAC),
        alpha.reshape(1, N_REAC),
        branching_ratio.reshape(1, N_REAC),
    )
